# baseline (speedup 1.0000x reference)
import jax
import jax.numpy as jnp
from jax import lax
from jax.experimental import pallas as pl
from jax.experimental.pallas import tpu as pltpu
from jax.experimental.pallas import tpu_sc as plsc

C = 256
N_ROIS = 1000
BINS = 49
TAPS = 16
N_CHUNKS = 8
CHUNK_USED = 2 * BINS
CHUNK_ROWS = 104
IDX_LEN = N_CHUNKS * CHUNK_ROWS


def _sc_body(table, roisf, out, rois_v, yc_v, xc_v, wy_v, wx_v,
             idxa_v, wa_v, idxb_v, wb_v, rows0, rows1, outa_v, outb_v,
             sem0, sem1, sem_oa, sem_ob):
    cid = lax.axis_index("c")
    sid = lax.axis_index("s")
    wid = sid * 2 + cid
    base = jnp.where(wid < 20, wid * 32, 640 + 30 * (wid - 20))
    pairs = jnp.where(wid < 20, 16, 15)

    pltpu.sync_copy(roisf, rois_v)

    iota = lax.iota(jnp.int32, 16)
    zeros_i = jnp.zeros((16,), jnp.int32)
    zeros_f = jnp.zeros((16,), jnp.float32)
    q = (iota >> 1).astype(jnp.float32) + (
        0.25 + 0.5 * (iota & 1).astype(jnp.float32))
    p_y = iota >> 2
    p_x = iota & 3
    tap_slot = 52 * iota - 3 * (iota & 1)
    ch_base = [(chn * 16 + iota) * BINS for chn in range(16)]

    for i in range(IDX_LEN // 16):
        idxa_v[pl.ds(i * 16, 16)] = zeros_i
        idxb_v[pl.ds(i * 16, 16)] = zeros_i

    def build_idx(g, idx_ref, w_ref):
        def splat(off):
            return plsc.load_gather(rois_v, [zeros_i + (g * 4 + off)])

        x1 = splat(0)
        y1 = splat(1)
        x2 = splat(2)
        y2 = splat(3)
        area = (y2 - y1) * (x2 - x1)
        k = ((area >= 12544.0).astype(jnp.int32)
             + (area >= 50176.0).astype(jnp.int32)
             + (area >= 200704.0).astype(jnp.int32))
        s_i = 256 >> k
        sf = s_i.astype(jnp.float32)
        scale = sf * (1.0 / 1024.0)
        boff = jnp.where(k == 0, 0,
                         jnp.where(k == 1, 65536,
                                   jnp.where(k == 2, 81920, 86016)))

        def axis_build(lo_img, hi_img, c_ref, wref):
            lo = lo_img * scale
            hi = hi_img * scale
            ln = jnp.maximum(hi - lo, 1.0)
            bsz = ln / 7.0
            gs = lo + q * bsz
            valid = (gs >= -1.0) & (gs <= sf)
            xx = jnp.maximum(gs, 0.0)
            fx = xx.astype(jnp.int32).astype(jnp.float32)
            clo = jnp.minimum(fx, sf - 1.0)
            xef = jnp.where(fx >= sf - 1.0, sf - 1.0, xx)
            chi = jnp.minimum(clo + 1.0, sf - 1.0)
            lw = xef - clo
            hw = 1.0 - lw
            plsc.store_scatter(c_ref, [2 * iota], clo.astype(jnp.int32))
            plsc.store_scatter(c_ref, [2 * iota + 1], chi.astype(jnp.int32))
            plsc.store_scatter(wref, [2 * iota], jnp.where(valid, hw, 0.0))
            plsc.store_scatter(wref, [2 * iota + 1], jnp.where(valid, lw, 0.0))

        axis_build(y1, y2, yc_v, wy_v)
        axis_build(x1, x2, xc_v, wx_v)

        def per_bin(b, _):
            oh = b // 7
            ow = b - oh * 7
            ysel = plsc.load_gather(yc_v, [p_y + 4 * oh])
            xsel = plsc.load_gather(xc_v, [p_x + 4 * ow])
            wys = plsc.load_gather(wy_v, [p_y + 4 * oh])
            wxs = plsc.load_gather(wx_v, [p_x + 4 * ow])
            idx16 = boff + ysel * s_i + xsel
            w16 = (0.25 * wys) * wxs
            plsc.store_scatter(idx_ref, [tap_slot + b], idx16)
            plsc.store_scatter(w_ref, [tap_slot + b], w16)
            return 0

        lax.fori_loop(0, BINS, per_bin, 0)

    bufs = (rows0, rows1)
    sems = (sem0, sem1)

    def chunk_desc(idx_ref, c):
        return pltpu.make_async_copy(
            table.at[idx_ref.at[pl.ds(c * CHUNK_ROWS, CHUNK_ROWS)]],
            bufs[c % 2], sems[c % 2])

    def acc_chunk(c, w_ref, outt_v):
        rbuf = bufs[c % 2]

        def bin_body(b, _):
            for tl in range(2):
                t = 2 * c + tl
                r = tl * BINS + b
                w = plsc.load_gather(
                    w_ref, [zeros_i + (c * CHUNK_ROWS + tl * BINS + b)])
                for chn in range(16):
                    val = w * rbuf[r, pl.ds(chn * 16, 16)]
                    if t == 0:
                        plsc.store_scatter(outt_v, [ch_base[chn] + b], val)
                    else:
                        plsc.addupdate_scatter(outt_v, [ch_base[chn] + b], val)
            return 0

        lax.fori_loop(0, BINS, bin_body, 0)

    def out_desc(outt_v, g, sem):
        return pltpu.make_async_copy(outt_v, out.at[g], sem)

    build_idx(base, idxa_v, wa_v)
    chunk_desc(idxa_v, 0).start()
    chunk_desc(idxa_v, 1).start()

    def pair_body(j, _):
        roi_a = base + 2 * j
        roi_b = roi_a + 1

        def phase(roi_cur, idx_cur, w_cur, outt_v, sem_out,
                  roi_nxt, idx_nxt, w_nxt, idx_follow, first_wait):
            @pl.when(jnp.logical_or(j != 0, first_wait))
            def _():
                out_desc(outt_v, roi_cur, sem_out).wait()

            for c in range(N_CHUNKS):
                chunk_desc(idx_cur, c).wait()
                acc_chunk(c, w_cur, outt_v)
                if c == 3:
                    build_idx(roi_nxt, idx_nxt, w_nxt)
                if c <= 5:
                    chunk_desc(idx_cur, c + 2).start()
                elif c == 6:
                    chunk_desc(idx_follow, 0).start()
                else:
                    chunk_desc(idx_follow, 1).start()
            out_desc(outt_v, roi_cur, sem_out).start()

        phase(roi_a, idxa_v, wa_v, outa_v, sem_oa,
              roi_b, idxb_v, wb_v, idxb_v, jnp.bool_(False))
        nxt = jnp.minimum(roi_a + 2, jnp.int32(N_ROIS - 1))
        phase(roi_b, idxb_v, wb_v, outb_v, sem_ob,
              nxt, idxa_v, wa_v, idxa_v, jnp.bool_(False))
        return 0

    lax.fori_loop(0, pairs, pair_body, 0)

    chunk_desc(idxa_v, 0).wait()
    chunk_desc(idxa_v, 1).wait()
    out_desc(outa_v, base, sem_oa).wait()
    out_desc(outb_v, base, sem_ob).wait()


@jax.jit
def _run(table, roisf):
    mesh = plsc.VectorSubcoreMesh(core_axis_name="c", subcore_axis_name="s")
    f = pl.kernel(
        _sc_body,
        out_type=jax.ShapeDtypeStruct((N_ROIS, C * BINS), jnp.float32),
        mesh=mesh,
        scratch_types=[
            pltpu.VMEM((N_ROIS * 4,), jnp.float32),
            pltpu.VMEM((32,), jnp.int32),
            pltpu.VMEM((32,), jnp.int32),
            pltpu.VMEM((32,), jnp.float32),
            pltpu.VMEM((32,), jnp.float32),
            pltpu.VMEM((IDX_LEN,), jnp.int32),
            pltpu.VMEM((IDX_LEN,), jnp.float32),
            pltpu.VMEM((IDX_LEN,), jnp.int32),
            pltpu.VMEM((IDX_LEN,), jnp.float32),
            pltpu.VMEM((CHUNK_ROWS, C), jnp.float32),
            pltpu.VMEM((CHUNK_ROWS, C), jnp.float32),
            pltpu.VMEM((C * BINS,), jnp.float32),
            pltpu.VMEM((C * BINS,), jnp.float32),
            pltpu.SemaphoreType.DMA,
            pltpu.SemaphoreType.DMA,
            pltpu.SemaphoreType.DMA,
            pltpu.SemaphoreType.DMA,
        ],
        compiler_params=pltpu.CompilerParams(needs_layout_passes=False),
    )
    return f(table, roisf)


def kernel(feat_p2, feat_p3, feat_p4, feat_p5, rois):
    tabs = []
    for f in (feat_p2, feat_p3, feat_p4, feat_p5):
        s = f.shape[-1]
        tabs.append(jnp.transpose(f[0], (1, 2, 0)).reshape(s * s, C))
    table = jnp.concatenate(tabs, axis=0)
    out = _run(table, rois.reshape(-1))
    return out.reshape(N_ROIS, C, 7, 7)


# speedup vs baseline: 3.6074x; 3.6074x over previous
import jax
import jax.numpy as jnp
from jax import lax
from jax.experimental import pallas as pl
from jax.experimental.pallas import tpu as pltpu
from jax.experimental.pallas import tpu_sc as plsc

C = 256
N_ROIS = 1000
BINS = 49
TAPS = 16
CHUNK_BINS = 7
CHUNK_ROWS = CHUNK_BINS * TAPS
N_CHUNKS = BINS // CHUNK_BINS


def _sc_body(table, roisf, out, rois_v, yc_v, xc_v, wy_v, wx_v,
             idxa_v, wa_v, idxb_v, wb_v, rows0, rows1, outa_v, outb_v,
             sem0, sem1, sem_oa, sem_ob):
    cid = lax.axis_index("c")
    sid = lax.axis_index("s")
    wid = sid * 2 + cid
    base = jnp.where(wid < 20, wid * 32, 640 + 30 * (wid - 20))
    pairs = jnp.where(wid < 20, 16, 15)

    pltpu.sync_copy(roisf, rois_v)

    iota = lax.iota(jnp.int32, 16)
    zeros_i = jnp.zeros((16,), jnp.int32)
    zeros_f = jnp.zeros((16,), jnp.float32)
    q = (iota >> 1).astype(jnp.float32) + (
        0.25 + 0.5 * (iota & 1).astype(jnp.float32))
    p_y = iota >> 2
    p_x = iota & 3
    ch_base = [(chn * 16 + iota) * BINS for chn in range(16)]

    def build_idx(g, idx_ref, w_ref):
        def splat(off):
            return plsc.load_gather(rois_v, [zeros_i + (g * 4 + off)])

        x1 = splat(0)
        y1 = splat(1)
        x2 = splat(2)
        y2 = splat(3)
        area = (y2 - y1) * (x2 - x1)
        k = ((area >= 12544.0).astype(jnp.int32)
             + (area >= 50176.0).astype(jnp.int32)
             + (area >= 200704.0).astype(jnp.int32))
        s_i = 256 >> k
        sf = s_i.astype(jnp.float32)
        scale = sf * (1.0 / 1024.0)
        boff = jnp.where(k == 0, 0,
                         jnp.where(k == 1, 65536,
                                   jnp.where(k == 2, 81920, 86016)))

        def axis_build(lo_img, hi_img, c_ref, wref):
            lo = lo_img * scale
            hi = hi_img * scale
            ln = jnp.maximum(hi - lo, 1.0)
            bsz = ln / 7.0
            gs = lo + q * bsz
            valid = (gs >= -1.0) & (gs <= sf)
            xx = jnp.maximum(gs, 0.0)
            fx = xx.astype(jnp.int32).astype(jnp.float32)
            clo = jnp.minimum(fx, sf - 1.0)
            xef = jnp.where(fx >= sf - 1.0, sf - 1.0, xx)
            chi = jnp.minimum(clo + 1.0, sf - 1.0)
            lw = xef - clo
            hw = 1.0 - lw
            plsc.store_scatter(c_ref, [2 * iota], clo.astype(jnp.int32))
            plsc.store_scatter(c_ref, [2 * iota + 1], chi.astype(jnp.int32))
            plsc.store_scatter(wref, [2 * iota], jnp.where(valid, hw, 0.0))
            plsc.store_scatter(wref, [2 * iota + 1], jnp.where(valid, lw, 0.0))

        axis_build(y1, y2, yc_v, wy_v)
        axis_build(x1, x2, xc_v, wx_v)

        def per_bin(b, _):
            oh = b // 7
            ow = b - oh * 7
            ysel = plsc.load_gather(yc_v, [p_y + 4 * oh])
            xsel = plsc.load_gather(xc_v, [p_x + 4 * ow])
            wys = plsc.load_gather(wy_v, [p_y + 4 * oh])
            wxs = plsc.load_gather(wx_v, [p_x + 4 * ow])
            idx16 = boff + ysel * s_i + xsel
            w16 = (0.25 * wys) * wxs
            plsc.store_scatter(idx_ref, [b * TAPS + iota], idx16)
            plsc.store_scatter(w_ref, [b * TAPS + iota], w16)
            return 0

        lax.fori_loop(0, BINS, per_bin, 0)

    bufs = (rows0, rows1)
    sems = (sem0, sem1)

    def chunk_desc(idx_ref, c, par):
        return pltpu.make_async_copy(
            table.at[idx_ref.at[pl.ds(c * CHUNK_ROWS, CHUNK_ROWS)]],
            bufs[par], sems[par])

    def acc_chunk(c, par, w_ref, outt_v):
        rbuf = bufs[par]

        def bin_body(bl, _):
            b = c * CHUNK_BINS + bl

            def tap_quad(t2, accs):
                accs = list(accs)
                for tu in range(8):
                    t = t2 * 8 + tu
                    w = plsc.load_gather(w_ref, [zeros_i + (b * TAPS + t)])
                    r = bl * TAPS + t
                    for chn in range(16):
                        accs[chn] = accs[chn] + w * rbuf[r, pl.ds(chn * 16, 16)]
                return tuple(accs)

            accs = lax.fori_loop(0, 2, tap_quad, tuple([zeros_f] * 16))
            for chn in range(16):
                plsc.store_scatter(outt_v, [ch_base[chn] + b], accs[chn])
            return 0

        lax.fori_loop(0, CHUNK_BINS, bin_body, 0)

    def out_desc(outt_v, g, sem):
        return pltpu.make_async_copy(outt_v, out.at[g], sem)

    build_idx(base, idxa_v, wa_v)
    chunk_desc(idxa_v, 0, 0).start()
    chunk_desc(idxa_v, 1, 1).start()

    def pair_body(j, _):
        roi_a = base + 2 * j
        roi_b = roi_a + 1
        nxt = jnp.minimum(roi_a + 2, jnp.int32(N_ROIS - 1))

        def phase(roi_cur, idx_cur, w_cur, outt_v, sem_out, poff,
                  roi_nxt, idx_nxt, w_nxt, idx_follow):
            @pl.when(j != 0)
            def _():
                out_desc(outt_v, roi_cur, sem_out).wait()

            for c in range(N_CHUNKS):
                par = (c + poff) % 2
                chunk_desc(idx_cur, c, par).wait()
                acc_chunk(c, par, w_cur, outt_v)
                if c == 3:
                    build_idx(roi_nxt, idx_nxt, w_nxt)
                if c <= 4:
                    chunk_desc(idx_cur, c + 2, par).start()
                elif c == 5:
                    chunk_desc(idx_follow, 0, (poff + 1) % 2).start()
                else:
                    chunk_desc(idx_follow, 1, poff).start()
            out_desc(outt_v, roi_cur, sem_out).start()

        phase(roi_a, idxa_v, wa_v, outa_v, sem_oa, 0,
              roi_b, idxb_v, wb_v, idxb_v)
        phase(roi_b, idxb_v, wb_v, outb_v, sem_ob, 1,
              nxt, idxa_v, wa_v, idxa_v)
        return 0

    lax.fori_loop(0, pairs, pair_body, 0)

    chunk_desc(idxa_v, 0, 0).wait()
    chunk_desc(idxa_v, 1, 1).wait()
    out_desc(outa_v, base, sem_oa).wait()
    out_desc(outb_v, base, sem_ob).wait()


@jax.jit
def _run(table, roisf):
    mesh = plsc.VectorSubcoreMesh(core_axis_name="c", subcore_axis_name="s")
    f = pl.kernel(
        _sc_body,
        out_type=jax.ShapeDtypeStruct((N_ROIS, C * BINS), jnp.float32),
        mesh=mesh,
        scratch_types=[
            pltpu.VMEM((N_ROIS * 4,), jnp.float32),
            pltpu.VMEM((32,), jnp.int32),
            pltpu.VMEM((32,), jnp.int32),
            pltpu.VMEM((32,), jnp.float32),
            pltpu.VMEM((32,), jnp.float32),
            pltpu.VMEM((BINS * TAPS,), jnp.int32),
            pltpu.VMEM((BINS * TAPS,), jnp.float32),
            pltpu.VMEM((BINS * TAPS,), jnp.int32),
            pltpu.VMEM((BINS * TAPS,), jnp.float32),
            pltpu.VMEM((CHUNK_ROWS, C), jnp.float32),
            pltpu.VMEM((CHUNK_ROWS, C), jnp.float32),
            pltpu.VMEM((C * BINS,), jnp.float32),
            pltpu.VMEM((C * BINS,), jnp.float32),
            pltpu.SemaphoreType.DMA,
            pltpu.SemaphoreType.DMA,
            pltpu.SemaphoreType.DMA,
            pltpu.SemaphoreType.DMA,
        ],
        compiler_params=pltpu.CompilerParams(needs_layout_passes=False),
    )
    return f(table, roisf)


def kernel(feat_p2, feat_p3, feat_p4, feat_p5, rois):
    tabs = []
    for f in (feat_p2, feat_p3, feat_p4, feat_p5):
        s = f.shape[-1]
        tabs.append(jnp.transpose(f[0], (1, 2, 0)).reshape(s * s, C))
    table = jnp.concatenate(tabs, axis=0)
    out = _run(table, rois.reshape(-1))
    return out.reshape(N_ROIS, C, 7, 7)
